# Initial kernel scaffold; baseline (speedup 1.0000x reference)
import jax
import jax.numpy as jnp
from jax.experimental import pallas as pl


def kernel(x, wg, fc1_w, fc2_w):
    raise NotImplementedError("write your pallas kernel here")


# speedup vs baseline: 8.9355x; 8.9355x over previous
import functools

import jax
import jax.numpy as jnp
from jax.experimental import pallas as pl

_B, _S, _D = 2, 2048, 768
_E, _K = 64, 8
_H = 48
_DFF = _E * _H
_TOK_BLK = 512


def _ffn_body(x_ref, wgt_ref, w1_ref, w2_ref, exp_ref, o_ref):
    xb = x_ref[...]
    g = jnp.dot(xb, wgt_ref[...], preferred_element_type=jnp.float32)
    gj = g[:, None, :]
    ge = g[:, :, None]
    jidx = jax.lax.broadcasted_iota(jnp.int32, (1, _E, _E), 2)
    eidx = jax.lax.broadcasted_iota(jnp.int32, (1, _E, _E), 1)
    beats = (gj > ge) | ((gj == ge) & (jidx < eidx))
    rank = jnp.sum(beats.astype(jnp.float32), axis=2)
    mask = (rank < _K).astype(jnp.float32)
    mexp = jnp.dot(mask, exp_ref[...], preferred_element_type=jnp.float32)
    h = jnp.maximum(
        jnp.dot(xb, w1_ref[...], preferred_element_type=jnp.float32), 0.0)
    o_ref[...] = jnp.dot(h * mexp, w2_ref[...],
                         preferred_element_type=jnp.float32)


@functools.partial(jax.jit, static_argnames=())
def kernel(x, wg, fc1_w, fc2_w):
    b, s, d = x.shape
    n = b * s
    xf = x.reshape(n, d)
    wgt = wg.T
    w1 = fc1_w.transpose(2, 0, 1).reshape(d, _DFF)
    w2 = fc2_w.transpose(0, 2, 1).reshape(_DFF, _D)
    expand = jnp.repeat(jnp.eye(_E, dtype=jnp.float32), _H, axis=1)

    yf = pl.pallas_call(
        _ffn_body,
        grid=(n // _TOK_BLK,),
        in_specs=[
            pl.BlockSpec((_TOK_BLK, d), lambda i: (i, 0)),
            pl.BlockSpec((d, _E), lambda i: (0, 0)),
            pl.BlockSpec((d, _DFF), lambda i: (0, 0)),
            pl.BlockSpec((_DFF, _D), lambda i: (0, 0)),
            pl.BlockSpec((_E, _DFF), lambda i: (0, 0)),
        ],
        out_specs=pl.BlockSpec((_TOK_BLK, _D), lambda i: (i, 0)),
        out_shape=jax.ShapeDtypeStruct((n, _D), jnp.float32),
    )(xf, wgt, w1, w2, expand)
    return yf.reshape(b, s, _D)
